# baseline (speedup 1.0000x reference)
import functools

import jax
import jax.numpy as jnp
from jax import lax
from jax.experimental import pallas as pl
from jax.experimental.pallas import tpu as pltpu
from jax.experimental.pallas import tpu_sc as plsc

B, L, D = 4096, 200, 64
WINDOW = 4
LANES = 16
DC = D // LANES
UNROLL = 4

NC, NS = 2, 16
NW = NC * NS
ROWS_PER_W = B // NW
TOK_PER_W = ROWS_PER_W * L
R = 2
CHUNK_T = R * L
N_CH = ROWS_PER_W // R
SUBS = [(o, min(128, CHUNK_T - o)) for o in range(0, CHUNK_T, 128)]


def _sc_body(tok_hbm, emb_hbm, out_hbm, idx_all, i0, i1, o0, o1, gsem, wsem):
    ins = (i0, i1)
    outs = (o0, o1)
    wid = lax.axis_index("c") * NS + lax.axis_index("s")
    wbase = pl.multiple_of(wid * TOK_PER_W, 8)
    pltpu.sync_copy(tok_hbm.at[pl.ds(wbase, TOK_PER_W)], idx_all)

    def fire_gather(c, b):
        ioff = pl.multiple_of(c * CHUNK_T, 8)
        for off, sz in SUBS:
            pltpu.async_copy(
                emb_hbm.at[idx_all.at[pl.ds(ioff + off, sz)]],
                ins[b].at[pl.ds(off, sz)],
                gsem.at[b],
            )

    def wait_gather(b):
        pltpu.make_async_copy(
            emb_hbm.at[pl.ds(0, CHUNK_T)], ins[b], gsem.at[b]
        ).wait()

    def fire_wb(c, b):
        off = pl.multiple_of(wbase + c * CHUNK_T, 8)
        pltpu.async_copy(outs[b], out_hbm.at[pl.ds(off, CHUNK_T)], wsem.at[b])

    def wait_wb(b):
        pltpu.make_async_copy(
            outs[b], out_hbm.at[pl.ds(0, CHUNK_T)], wsem.at[b]
        ).wait()

    def compute(b):
        src = ins[b]
        dst = outs[b]
        zero = jnp.zeros((LANES,), jnp.float32)
        for r in range(R):
            init = tuple((zero, zero, zero) for _ in range(DC))

            @plsc.parallel_loop(0, L, 1, unroll=UNROLL, carry=init)
            def _loop(l, carry, r=r):
                row = r * L + l
                nxt = []
                for dc in range(DC):
                    s1, s2, s3 = carry[dc]
                    e = src[row, pl.ds(dc * LANES, LANES)]
                    o = s3 + e
                    dst[row, pl.ds(dc * LANES, LANES)] = o
                    ne = o - s3
                    nxt.append((ne, s1 + ne, s2 + ne))
                return tuple(nxt)

    fire_gather(0, 0)

    def iter_body(i, _):
        for j in range(2):
            c = 2 * i + j
            b = j
            wait_gather(b)
            if j == 0:
                fire_gather(c + 1, 1 - b)

                @pl.when(i > 0)
                def _():
                    wait_wb(b)
            else:
                @pl.when(i < N_CH // 2 - 1)
                def _():
                    fire_gather(c + 1, 1 - b)

                @pl.when(i > 0)
                def _():
                    wait_wb(b)

            compute(b)
            fire_wb(c, b)
        return 0

    lax.fori_loop(0, N_CH // 2, iter_body, 0)
    wait_wb(0)
    wait_wb(1)


@jax.jit
def _sc_call(tok, embedding):
    mesh = plsc.VectorSubcoreMesh(
        core_axis_name="c", subcore_axis_name="s", num_cores=NC, num_subcores=NS
    )
    f = pl.kernel(
        _sc_body,
        out_type=jax.ShapeDtypeStruct((B * L, D), jnp.float32),
        mesh=mesh,
        scratch_types=[
            pltpu.VMEM((TOK_PER_W,), jnp.int32),
            pltpu.VMEM((CHUNK_T, D), jnp.float32),
            pltpu.VMEM((CHUNK_T, D), jnp.float32),
            pltpu.VMEM((CHUNK_T, D), jnp.float32),
            pltpu.VMEM((CHUNK_T, D), jnp.float32),
            pltpu.SemaphoreType.DMA((2,)),
            pltpu.SemaphoreType.DMA((2,)),
        ],
        compiler_params=pltpu.CompilerParams(use_tc_tiling_on_sc=False),
    )
    return f(tok, embedding)


def kernel(tokens, embedding):
    tok = tokens.reshape(-1).astype(jnp.int32)
    out = _sc_call(tok, embedding)
    return out.reshape(B, L, D)


# speedup vs baseline: 1.1031x; 1.1031x over previous
import functools

import jax
import jax.numpy as jnp
from jax import lax
from jax.experimental import pallas as pl
from jax.experimental.pallas import tpu as pltpu
from jax.experimental.pallas import tpu_sc as plsc

B, L, D = 4096, 200, 64
DP = 128
LANES = 16
DC = D // LANES
UNROLL = 4

NC, NS = 2, 16
NW = NC * NS
ROWS_PER_W = B // NW
TOK_PER_W = ROWS_PER_W * L
NIB = 2
NOB = 2
NXB = 3
MAIN = ROWS_PER_W - 2
STEP = 6
SUBS = [(0, 128), (128, 72)]


def _sc_body(tok_hbm, emb_hbm, out_hbm, x0, x1, x2, i0, i1, o0, o1,
             gsem, wsem, xsem):
    idxs = (x0, x1, x2)
    ins = (i0, i1)
    outs = (o0, o1)
    wid = lax.axis_index("c") * NS + lax.axis_index("s")
    wbase = pl.multiple_of(wid * TOK_PER_W, 8)
    brow0 = wid * ROWS_PER_W

    def fire_idx(k, bx):
        pltpu.async_copy(
            tok_hbm.at[pl.ds(wbase + pl.multiple_of(k * L, 8), L)],
            idxs[bx],
            xsem.at[bx],
        )

    def wait_idx(bx):
        pltpu.make_async_copy(
            tok_hbm.at[pl.ds(0, L)], idxs[bx], xsem.at[bx]
        ).wait()

    def fire_gather(k, bi, bx):
        for off, sz in SUBS:
            pltpu.async_copy(
                emb_hbm.at[idxs[bx].at[pl.ds(off, sz)]],
                ins[bi].at[pl.ds(off, sz)],
                gsem.at[bi],
            )

    def wait_gather(bi):
        pltpu.make_async_copy(
            emb_hbm.at[pl.ds(0, L)], ins[bi], gsem.at[bi]
        ).wait()

    def fire_wb(k, bo):
        pltpu.async_copy(
            outs[bo], out_hbm.at[pl.ds(brow0 + k, 1)], wsem.at[bo]
        )

    def wait_wb(bo):
        pltpu.make_async_copy(
            outs[bo], out_hbm.at[pl.ds(0, 1)], wsem.at[bo]
        ).wait()

    def compute(bi, bo):
        src = ins[bi]
        dst = outs[bo]
        zero = jnp.zeros((LANES,), jnp.float32)
        init = tuple((zero, zero, zero) for _ in range(DC))

        @plsc.parallel_loop(0, L, 1, unroll=UNROLL, carry=init)
        def _loop(l, carry):
            nxt = []
            for dc in range(DC):
                s1, s2, s3 = carry[dc]
                e = src[l, pl.ds(dc * LANES, LANES)]
                o = s3 + e
                dst[0, l, pl.ds(dc * LANES, LANES)] = o
                ne = o - s3
                nxt.append((ne, s1 + ne, s2 + ne))
            return tuple(nxt)

    fire_idx(0, 0)
    fire_idx(1, 1)
    wait_idx(0)
    fire_gather(0, 0, 0)

    def iter_body(i, _):
        for j in range(STEP):
            k = i * STEP + j
            bi = j % NIB
            bo = j % NOB
            wait_gather(bi)
            if j < 2:
                @pl.when(i > 0)
                def _():
                    wait_wb(bo)
            else:
                wait_wb(bo)
            wait_idx((j + 1) % NXB)
            fire_gather(k + 1, (j + 1) % NIB, (j + 1) % NXB)
            fire_idx(k + 2, (j + 2) % NXB)
            compute(bi, bo)
            fire_wb(k, bo)
        return 0

    lax.fori_loop(0, MAIN // STEP, iter_body, 0)

    for k in (MAIN, MAIN + 1):
        bi = k % NIB
        bo = k % NOB
        wait_gather(bi)
        wait_wb(bo)
        if k + 1 < ROWS_PER_W:
            wait_idx((k + 1) % NXB)
            fire_gather(k + 1, (k + 1) % NIB, (k + 1) % NXB)
        compute(bi, bo)
        fire_wb(k, bo)
    wait_wb(MAIN % NOB)
    wait_wb((MAIN + 1) % NOB)


@jax.jit
def _sc_call(tok, emb128):
    mesh = plsc.VectorSubcoreMesh(
        core_axis_name="c", subcore_axis_name="s", num_cores=NC, num_subcores=NS
    )
    f = pl.kernel(
        _sc_body,
        out_type=jax.ShapeDtypeStruct((B, L, D), jnp.float32),
        mesh=mesh,
        scratch_types=[
            pltpu.VMEM((L,), jnp.int32),
            pltpu.VMEM((L,), jnp.int32),
            pltpu.VMEM((L,), jnp.int32),
            pltpu.VMEM((L, DP), jnp.float32),
            pltpu.VMEM((L, DP), jnp.float32),
            pltpu.VMEM((1, L, D), jnp.float32),
            pltpu.VMEM((1, L, D), jnp.float32),
            pltpu.SemaphoreType.DMA((NIB,)),
            pltpu.SemaphoreType.DMA((NOB,)),
            pltpu.SemaphoreType.DMA((NXB,)),
        ],
        compiler_params=pltpu.CompilerParams(use_tc_tiling_on_sc=True),
    )
    return f(tok, emb128)


def kernel(tokens, embedding):
    tok = tokens.reshape(-1).astype(jnp.int32)
    emb128 = jnp.pad(embedding, ((0, 0), (0, DP - D)))
    return _sc_call(tok, emb128)
